# baseline (speedup 1.0000x reference)
import functools

import jax
import jax.numpy as jnp
from jax import lax
from jax.experimental import pallas as pl
from jax.experimental.pallas import tpu as pltpu
from jax.experimental.pallas import tpu_sc as plsc

N = 10000
E = 320000
D = 128
PE = 16
DPE = D + PE
K = 4
PE_ALPHA = 0.5
PE_BETA = 0.5

NC = 2
NS = 16
NW = NC * NS
EW = E // NW
C = 112
G = 90
EPW = G * C
EP = EPW * NW
NA = N + 16
DUMP = N + 8
SLAB = 624
TAIL_OFF = (NS - 1) * SLAB

BLK = 1000
NBLK = N // BLK

_f32 = jnp.float32


def _make_prop(with_h: bool):
  def body(*refs):
    if with_h:
      (tab144, rc3, z144, hp_out,
       acc144, ia, ib, ca, cb, r144a, r144b,
       sia, sib, sha, shb, ssa, ssb) = refs
      r144 = (r144a, r144b)
      colb = (ca, cb)
      semh = (sha, shb)
      sems_h = (ssa, ssb)
    else:
      (rc3, z16, pp_out, acc16, ia, ib, ones16v, sia, sib) = refs
    idx = (ia, ib)
    semi = (sia, sib)
    c = lax.axis_index("c")
    s = lax.axis_index("s")
    wid = c * NS + s

    def per_slab(fn, rows_total):
      @pl.when(s < NS - 1)
      def _():
        fn(pl.multiple_of(s * SLAB, 8), SLAB)

      @pl.when(s == NS - 1)
      def _():
        fn(TAIL_OFF, rows_total - TAIL_OFF)

    def zero(off, rows):
      if with_h:
        pltpu.sync_copy(z144.at[pl.ds(off, rows)], acc144.at[pl.ds(off, rows)])
      else:
        pltpu.sync_copy(z16.at[pl.ds(off, rows)], acc16.at[pl.ds(off, rows)])

    per_slab(zero, NA)
    if not with_h:
      def fill(r, carry):
        ones16v[r, :] = jnp.full((PE,), 1.0, _f32)
        return carry
      lax.fori_loop(0, C, fill, 0)
    plsc.subcore_barrier()

    def issue_idx(b, g):
      pltpu.async_copy(rc3.at[wid, g], idx[b], semi[b])

    def wait_idx(b):
      pltpu.make_async_copy(rc3.at[0, 0], idx[b], semi[b]).wait()

    pltpu.sync_copy(rc3.at[wid, 0], idx[0])
    if with_h:
      def issue_gather(b):
        pltpu.async_copy(tab144.at[idx[b].at[0]], r144[b], semh[b])

      def wait_gather(b):
        pltpu.make_async_copy(tab144.at[pl.ds(0, C)], r144[b], semh[b]).wait()

      def issue_scatter(b):
        for i in range(C // 16):
          colb[b][pl.ds(i * 16, 16)] = idx[b][1, pl.ds(i * 16, 16)]
        pltpu.async_copy(r144[b], acc144.at[colb[b]], sems_h[b], add=True)

      def wait_scatter(b):
        pltpu.make_async_copy(r144[b], acc144.at[pl.ds(0, C)],
                              sems_h[b]).wait()

      issue_gather(0)
    issue_idx(1, 1)

    def pair(gp, carry):
      for b in range(2):
        g = 2 * gp + b
        wait_idx(1 - b)
        if with_h:
          if b == 0:
            @pl.when(gp > 0)
            def _():
              wait_scatter(1)
          else:
            wait_scatter(0)
          issue_gather(1 - b)
          wait_gather(b)
          issue_scatter(b)
        else:
          pltpu.sync_copy(ones16v, acc16.at[idx[b].at[1]], add=True)
        issue_idx(b, jnp.minimum(g + 2, G - 1))
      return carry

    lax.fori_loop(0, G // 2, pair, 0)
    wait_idx(1)
    if with_h:
      wait_gather(0)
      wait_scatter(1)

    plsc.subcore_barrier()

    def flush(off, rows):
      ob = pl.multiple_of(c * N + off, 8)
      if with_h:
        pltpu.sync_copy(acc144.at[pl.ds(off, rows)], hp_out.at[pl.ds(ob, rows)])
      else:
        pltpu.sync_copy(acc16.at[pl.ds(off, rows)], pp_out.at[pl.ds(ob, rows)])

    per_slab(flush, N)

  if with_h:
    out_type = jax.ShapeDtypeStruct((NC * N, DPE), _f32)
    scratch = [pltpu.VMEM_SHARED((NA, DPE), _f32),
               pltpu.VMEM((2, C), jnp.int32),
               pltpu.VMEM((2, C), jnp.int32),
               pltpu.VMEM((C,), jnp.int32),
               pltpu.VMEM((C,), jnp.int32),
               pltpu.VMEM((C, DPE), _f32),
               pltpu.VMEM((C, DPE), _f32)] + [pltpu.SemaphoreType.DMA] * 6
  else:
    out_type = jax.ShapeDtypeStruct((NC * N, PE), _f32)
    scratch = [pltpu.VMEM_SHARED((NA, PE), _f32),
               pltpu.VMEM((2, C), jnp.int32),
               pltpu.VMEM((2, C), jnp.int32),
               pltpu.VMEM((C, PE), _f32),
               pltpu.SemaphoreType.DMA,
               pltpu.SemaphoreType.DMA]

  mesh = plsc.VectorSubcoreMesh(core_axis_name="c", subcore_axis_name="s",
                                num_cores=NC, num_subcores=NS)
  return pl.kernel(
      body, out_type=out_type, mesh=mesh, scratch_types=scratch,
      compiler_params=pltpu.CompilerParams(use_tc_tiling_on_sc=False))


_make_prop_cached = functools.cache(_make_prop)


def _prop_both(*args):
  return _make_prop_cached(True)(*args)


def _prop_pe(*args):
  return _make_prop_cached(False)(*args)


def _pro_body(x, w1, b1, w2, b2, pos, wpe, bpe, dp, w0, bc0, t0,
              h0_o, pe0_o, tab_o, dinv_o, hid_o):
  deg = 1.0 + dp[0][:, 0:1] + dp[1][:, 0:1]
  dinv = lax.rsqrt(deg)
  h = jnp.maximum(
      jnp.dot(x[...], w1[...], preferred_element_type=_f32) + b1[...], 0.0)
  h = jnp.dot(h, w2[...], preferred_element_type=_f32) + b2[...]
  pe = jnp.tanh(jnp.dot(pos[...], wpe[...], preferred_element_type=_f32)
                + bpe[...])
  gamma0 = t0[0, 0] * jnp.tanh(
      jnp.dot(pe, w0[...], preferred_element_type=_f32) + bc0[0, 0])
  h0_o[...] = h
  pe0_o[...] = pe
  tab_o[:, :D] = dinv * h
  tab_o[:, D:] = dinv * pe
  dinv_o[...] = dinv
  hid_o[...] = gamma0 * h


def _prologue(x, w1, b1, w2, b2, pos, wpe, bpe, dp, w0, bc0, t0):
  full = lambda *shape: pl.BlockSpec(shape, lambda i: (0,) * len(shape))
  rowblk = lambda d: pl.BlockSpec((BLK, d), lambda i: (i, 0))
  return pl.pallas_call(
      _pro_body,
      grid=(NBLK,),
      in_specs=[rowblk(D), full(D, D), full(1, D), full(D, D), full(1, D),
                rowblk(32), full(32, PE), full(1, PE),
                pl.BlockSpec((NC, BLK, PE), lambda i: (0, i, 0)),
                full(PE, 1), full(1, 1), full(1, 1)],
      out_specs=[rowblk(D), rowblk(PE), rowblk(DPE), rowblk(1),
                 rowblk(D)],
      out_shape=[jax.ShapeDtypeStruct((N, D), _f32),
                 jax.ShapeDtypeStruct((N, PE), _f32),
                 jax.ShapeDtypeStruct((N, DPE), _f32),
                 jax.ShapeDtypeStruct((N, 1), _f32),
                 jax.ShapeDtypeStruct((N, D), _f32)],
  )(x, w1, b1, w2, b2, pos, wpe, bpe, dp, w0, bc0, t0)


def _flash_body(pef, pe_b, wcor, bcor, corr_o):
  pe_full = pef[...]
  cl_full = jnp.dot(pe_full, wcor[...], preferred_element_type=_f32) + bcor[...]
  cl_b = jnp.dot(pe_b[...], wcor[...], preferred_element_type=_f32) + bcor[...]
  acc = jnp.zeros((BLK, PE), _f32)
  for j in range(NBLK):
    cl_j = cl_full[j * BLK:(j + 1) * BLK, :]
    pe_j = pe_full[j * BLK:(j + 1) * BLK, :]
    s = lax.dot_general(cl_b, cl_j, (((1,), (1,)), ((), ())),
                        preferred_element_type=_f32)
    p = 0.5 + 0.5 * jnp.tanh(0.5 * s)
    acc = acc + jnp.dot(p, pe_j, preferred_element_type=_f32)
  corr_o[...] = acc


def _flash_tc(pe, wcor, bcor):
  full = lambda *shape: pl.BlockSpec(shape, lambda i: (0,) * len(shape))
  rowblk = lambda d: pl.BlockSpec((BLK, d), lambda i: (i, 0))
  return pl.pallas_call(
      _flash_body,
      grid=(NBLK,),
      in_specs=[full(N, PE), rowblk(PE), full(PE, PE), full(1, PE)],
      out_specs=rowblk(PE),
      out_shape=jax.ShapeDtypeStruct((N, PE), _f32),
  )(pe, pe, wcor, bcor)


def _iter_body(pe_b, raw, h, hp, dinv, corr, wck, bck, tk, hid,
               h_new_o, pe_new_o, tab_o, hid_o):
  dv = dinv[...]
  dv2 = dv * dv
  hpc = hp[0] + hp[1]
  h_new = dv * hpc[:, :D] + dv2 * h[...]
  pe_tpo = dv * hpc[:, D:] + dv2 * pe_b[...]
  pe_mix = (1.0 + PE_BETA) * pe_tpo - PE_BETA * corr[...]
  pe_new = jnp.tanh(PE_ALPHA * raw[...] + (1.0 - PE_ALPHA) * pe_mix)
  gamma = tk[0, 0] * jnp.tanh(
      jnp.dot(pe_new, wck[...], preferred_element_type=_f32) + bck[0, 0])
  h_new_o[...] = h_new
  pe_new_o[...] = pe_new
  tab_o[:, :D] = dv * h_new
  tab_o[:, D:] = dv * pe_new
  hid_o[...] = hid[...] + gamma * h_new


def _iter_tc(pe, raw, h, hp, dinv, corr, wck, bck, tk, hid):
  full = lambda *shape: pl.BlockSpec(shape, lambda i: (0,) * len(shape))
  rowblk = lambda d: pl.BlockSpec((BLK, d), lambda i: (i, 0))
  return pl.pallas_call(
      _iter_body,
      grid=(NBLK,),
      in_specs=[rowblk(PE), rowblk(PE), rowblk(D),
                pl.BlockSpec((NC, BLK, DPE), lambda i: (0, i, 0)),
                rowblk(1), rowblk(PE), full(PE, 1),
                full(1, 1), full(1, 1), rowblk(D)],
      out_specs=[rowblk(D), rowblk(PE), rowblk(DPE), rowblk(D)],
      out_shape=[jax.ShapeDtypeStruct((N, D), _f32),
                 jax.ShapeDtypeStruct((N, PE), _f32),
                 jax.ShapeDtypeStruct((N, DPE), _f32),
                 jax.ShapeDtypeStruct((N, D), _f32)],
  )(pe, raw, h, hp, dinv, corr, wck, bck, tk, hid)


def kernel(node_feat, pos_enc, edge_index, W1, b1, W2, b2, Wpe, bpe, temp,
           Wcoef, bcoef, Wcor, bcor):
  row = edge_index[0]
  col = edge_index[1]
  padw = EPW - EW
  rowp = jnp.concatenate(
      [row.reshape(NW, EW), jnp.zeros((NW, padw), jnp.int32)], axis=1)
  dump_cols = jnp.broadcast_to(
      N + (jnp.arange(padw, dtype=jnp.int32) % (NA - N)), (NW, padw))
  colp = jnp.concatenate([col.reshape(NW, EW), dump_cols], axis=1)
  rc3 = jnp.stack(
      [rowp.reshape(NW, G, C), colp.reshape(NW, G, C)], axis=2)
  z144 = jnp.zeros((NA, DPE), _f32)
  z16 = jnp.zeros((NA, PE), _f32)

  dp = _prop_pe(rc3, z16).reshape(NC, N, PE)

  h, pe, tab, dinv, hidden = _prologue(
      node_feat, W1, b1.reshape(1, D), W2, b2.reshape(1, D),
      pos_enc, Wpe, bpe.reshape(1, PE), dp,
      Wcoef[0].reshape(PE, 1), bcoef[0].reshape(1, 1), temp[0].reshape(1, 1))
  raw_pe = pe

  for k in range(K):
    hp = _prop_both(tab, rc3, z144)
    corr = _flash_tc(pe, Wcor, bcor.reshape(1, PE))
    hp = hp.reshape(NC, N, DPE)
    h, pe, tab, hidden = _iter_tc(
        pe, raw_pe, h, hp, dinv, corr,
        Wcoef[k + 1].reshape(PE, 1), bcoef[k + 1].reshape(1, 1),
        temp[k + 1].reshape(1, 1), hidden)
  return hidden, pe


# speedup vs baseline: 1.0421x; 1.0421x over previous
import functools

import jax
import jax.numpy as jnp
from jax import lax
from jax.experimental import pallas as pl
from jax.experimental.pallas import tpu as pltpu
from jax.experimental.pallas import tpu_sc as plsc

N = 10000
E = 320000
D = 128
PE = 16
K = 4
PE_ALPHA = 0.5
PE_BETA = 0.5

NC = 2
NS = 16
NW = NC * NS
EW = E // NW
C = 112
G = 90
EPW = G * C
NA = N + 16
SLAB = 624
TAIL_OFF = (NS - 1) * SLAB

BLK = 1000
NBLK = N // BLK

_f32 = jnp.float32


def _make_prop(with_h: bool):
  def body(*refs):
    if with_h:
      (tab128, tab16, rc3, z128, z16, hp_out, pp_out,
       acc128, acc16, ia, ib, ca, cb, r128a, r128b, r16a, r16b,
       sia, sib, sha, shb, spa, spb, ssa, ssb, sta, stb) = refs
      r128 = (r128a, r128b)
      r16 = (r16a, r16b)
      colb = (ca, cb)
      semh = (sha, shb)
      semp = (spa, spb)
      sems_h = (ssa, ssb)
      sems_p = (sta, stb)
    else:
      (rc3, z16, pp_out, acc16, ia, ib, ones16v, sia, sib) = refs
    idx = (ia, ib)
    semi = (sia, sib)
    c = lax.axis_index("c")
    s = lax.axis_index("s")
    wid = c * NS + s

    def per_slab(fn, rows_total):
      @pl.when(s < NS - 1)
      def _():
        fn(pl.multiple_of(s * SLAB, 8), SLAB)

      @pl.when(s == NS - 1)
      def _():
        fn(TAIL_OFF, rows_total - TAIL_OFF)

    def zero(off, rows):
      if with_h:
        pltpu.sync_copy(z128.at[pl.ds(off, rows)], acc128.at[pl.ds(off, rows)])
      pltpu.sync_copy(z16.at[pl.ds(off, rows)], acc16.at[pl.ds(off, rows)])

    per_slab(zero, NA)
    if not with_h:
      def fill(r, carry):
        ones16v[r, :] = jnp.full((PE,), 1.0, _f32)
        return carry
      lax.fori_loop(0, C, fill, 0)
    plsc.subcore_barrier()

    def issue_idx(b, g):
      pltpu.async_copy(rc3.at[wid, g], idx[b], semi[b])

    def wait_idx(b):
      pltpu.make_async_copy(rc3.at[0, 0], idx[b], semi[b]).wait()

    pltpu.sync_copy(rc3.at[wid, 0], idx[0])
    if with_h:
      def issue_gather(b):
        pltpu.async_copy(tab128.at[idx[b].at[0]], r128[b], semh[b])
        pltpu.async_copy(tab16.at[idx[b].at[0]], r16[b], semp[b])

      def wait_gather(b):
        pltpu.make_async_copy(tab128.at[pl.ds(0, C)], r128[b], semh[b]).wait()
        pltpu.make_async_copy(tab16.at[pl.ds(0, C)], r16[b], semp[b]).wait()

      def issue_scatter(b):
        for i in range(C // 16):
          colb[b][pl.ds(i * 16, 16)] = idx[b][1, pl.ds(i * 16, 16)]
        pltpu.async_copy(r128[b], acc128.at[colb[b]], sems_h[b], add=True)
        pltpu.async_copy(r16[b], acc16.at[colb[b]], sems_p[b], add=True)

      def wait_scatter(b):
        pltpu.make_async_copy(r128[b], acc128.at[pl.ds(0, C)],
                              sems_h[b]).wait()
        pltpu.make_async_copy(r16[b], acc16.at[pl.ds(0, C)],
                              sems_p[b]).wait()

      issue_gather(0)
    issue_idx(1, 1)

    def pair(gp, carry):
      for b in range(2):
        g = 2 * gp + b
        wait_idx(1 - b)
        if with_h:
          if b == 0:
            @pl.when(gp > 0)
            def _():
              wait_scatter(1)
          else:
            wait_scatter(0)
          issue_gather(1 - b)
          wait_gather(b)
          issue_scatter(b)
        else:
          pltpu.sync_copy(ones16v, acc16.at[idx[b].at[1]], add=True)
        issue_idx(b, jnp.minimum(g + 2, G - 1))
      return carry

    lax.fori_loop(0, G // 2, pair, 0)
    wait_idx(1)
    if with_h:
      wait_gather(0)
      wait_scatter(1)

    plsc.subcore_barrier()

    def flush(off, rows):
      ob = pl.multiple_of(c * N + off, 8)
      if with_h:
        pltpu.sync_copy(acc128.at[pl.ds(off, rows)], hp_out.at[pl.ds(ob, rows)])
      pltpu.sync_copy(acc16.at[pl.ds(off, rows)], pp_out.at[pl.ds(ob, rows)])

    per_slab(flush, N)

  if with_h:
    out_type = [jax.ShapeDtypeStruct((NC * N, D), _f32),
                jax.ShapeDtypeStruct((NC * N, PE), _f32)]
    scratch = [pltpu.VMEM_SHARED((NA, D), _f32),
               pltpu.VMEM_SHARED((NA, PE), _f32),
               pltpu.VMEM((2, C), jnp.int32),
               pltpu.VMEM((2, C), jnp.int32),
               pltpu.VMEM((C,), jnp.int32),
               pltpu.VMEM((C,), jnp.int32),
               pltpu.VMEM((C, D), _f32),
               pltpu.VMEM((C, D), _f32),
               pltpu.VMEM((C, PE), _f32),
               pltpu.VMEM((C, PE), _f32)] + [pltpu.SemaphoreType.DMA] * 10
  else:
    out_type = jax.ShapeDtypeStruct((NC * N, PE), _f32)
    scratch = [pltpu.VMEM_SHARED((NA, PE), _f32),
               pltpu.VMEM((2, C), jnp.int32),
               pltpu.VMEM((2, C), jnp.int32),
               pltpu.VMEM((C, PE), _f32),
               pltpu.SemaphoreType.DMA,
               pltpu.SemaphoreType.DMA]

  mesh = plsc.VectorSubcoreMesh(core_axis_name="c", subcore_axis_name="s",
                                num_cores=NC, num_subcores=NS)
  return pl.kernel(
      body, out_type=out_type, mesh=mesh, scratch_types=scratch,
      compiler_params=pltpu.CompilerParams(use_tc_tiling_on_sc=False))


_make_prop_cached = functools.cache(_make_prop)


def _prop_both(*args):
  return _make_prop_cached(True)(*args)


def _prop_pe(*args):
  return _make_prop_cached(False)(*args)


def _pro_body(x, w1, b1, w2, b2, pos, wpe, bpe, dp, w0, bc0, t0,
              h0_o, pe0_o, hs_o, pes_o, dinv_o, hid_o):
  deg = 1.0 + dp[0][:, 0:1] + dp[1][:, 0:1]
  dinv = lax.rsqrt(deg)
  h = jnp.maximum(
      jnp.dot(x[...], w1[...], preferred_element_type=_f32) + b1[...], 0.0)
  h = jnp.dot(h, w2[...], preferred_element_type=_f32) + b2[...]
  pe = jnp.tanh(jnp.dot(pos[...], wpe[...], preferred_element_type=_f32)
                + bpe[...])
  gamma0 = t0[0, 0] * jnp.tanh(
      jnp.dot(pe, w0[...], preferred_element_type=_f32) + bc0[0, 0])
  h0_o[...] = h
  pe0_o[...] = pe
  hs_o[...] = dinv * h
  pes_o[...] = dinv * pe
  dinv_o[...] = dinv
  hid_o[...] = gamma0 * h


def _prologue(x, w1, b1, w2, b2, pos, wpe, bpe, dp, w0, bc0, t0):
  full = lambda *shape: pl.BlockSpec(shape, lambda i: (0,) * len(shape))
  rowblk = lambda d: pl.BlockSpec((BLK, d), lambda i: (i, 0))
  return pl.pallas_call(
      _pro_body,
      grid=(NBLK,),
      in_specs=[rowblk(D), full(D, D), full(1, D), full(D, D), full(1, D),
                rowblk(32), full(32, PE), full(1, PE),
                pl.BlockSpec((NC, BLK, PE), lambda i: (0, i, 0)),
                full(PE, 1), full(1, 1), full(1, 1)],
      out_specs=[rowblk(D), rowblk(PE), rowblk(D), rowblk(PE), rowblk(1),
                 rowblk(D)],
      out_shape=[jax.ShapeDtypeStruct((N, D), _f32),
                 jax.ShapeDtypeStruct((N, PE), _f32),
                 jax.ShapeDtypeStruct((N, D), _f32),
                 jax.ShapeDtypeStruct((N, PE), _f32),
                 jax.ShapeDtypeStruct((N, 1), _f32),
                 jax.ShapeDtypeStruct((N, D), _f32)],
  )(x, w1, b1, w2, b2, pos, wpe, bpe, dp, w0, bc0, t0)


def _flash_body(pef, pe_b, wcor, bcor, corr_o):
  pe_full = pef[...]
  cl_full = jnp.dot(pe_full, wcor[...], preferred_element_type=_f32) + bcor[...]
  cl_b = jnp.dot(pe_b[...], wcor[...], preferred_element_type=_f32) + bcor[...]
  acc = jnp.zeros((BLK, PE), _f32)
  for j in range(NBLK):
    cl_j = cl_full[j * BLK:(j + 1) * BLK, :]
    pe_j = pe_full[j * BLK:(j + 1) * BLK, :]
    s = lax.dot_general(cl_b, cl_j, (((1,), (1,)), ((), ())),
                        preferred_element_type=_f32)
    p = 0.5 + 0.5 * jnp.tanh(0.5 * s)
    acc = acc + jnp.dot(p, pe_j, preferred_element_type=_f32)
  corr_o[...] = acc


def _flash_tc(pe, wcor, bcor):
  full = lambda *shape: pl.BlockSpec(shape, lambda i: (0,) * len(shape))
  rowblk = lambda d: pl.BlockSpec((BLK, d), lambda i: (i, 0))
  return pl.pallas_call(
      _flash_body,
      grid=(NBLK,),
      in_specs=[full(N, PE), rowblk(PE), full(PE, PE), full(1, PE)],
      out_specs=rowblk(PE),
      out_shape=jax.ShapeDtypeStruct((N, PE), _f32),
  )(pe, pe, wcor, bcor)


def _iter_body(pe_b, raw, h, hp, pp, dinv, corr, wck, bck, tk, hid,
               h_new_o, pe_new_o, hs_o, pes_o, hid_o):
  dv = dinv[...]
  dv2 = dv * dv
  h_new = dv * (hp[0] + hp[1]) + dv2 * h[...]
  pe_tpo = dv * (pp[0] + pp[1]) + dv2 * pe_b[...]
  pe_mix = (1.0 + PE_BETA) * pe_tpo - PE_BETA * corr[...]
  pe_new = jnp.tanh(PE_ALPHA * raw[...] + (1.0 - PE_ALPHA) * pe_mix)
  gamma = tk[0, 0] * jnp.tanh(
      jnp.dot(pe_new, wck[...], preferred_element_type=_f32) + bck[0, 0])
  h_new_o[...] = h_new
  pe_new_o[...] = pe_new
  hs_o[...] = dv * h_new
  pes_o[...] = dv * pe_new
  hid_o[...] = hid[...] + gamma * h_new


def _iter_tc(pe, raw, h, hp, pp, dinv, corr, wck, bck, tk, hid):
  full = lambda *shape: pl.BlockSpec(shape, lambda i: (0,) * len(shape))
  rowblk = lambda d: pl.BlockSpec((BLK, d), lambda i: (i, 0))
  return pl.pallas_call(
      _iter_body,
      grid=(NBLK,),
      in_specs=[rowblk(PE), rowblk(PE), rowblk(D),
                pl.BlockSpec((NC, BLK, D), lambda i: (0, i, 0)),
                pl.BlockSpec((NC, BLK, PE), lambda i: (0, i, 0)),
                rowblk(1), rowblk(PE), full(PE, 1),
                full(1, 1), full(1, 1), rowblk(D)],
      out_specs=[rowblk(D), rowblk(PE), rowblk(D), rowblk(PE), rowblk(D)],
      out_shape=[jax.ShapeDtypeStruct((N, D), _f32),
                 jax.ShapeDtypeStruct((N, PE), _f32),
                 jax.ShapeDtypeStruct((N, D), _f32),
                 jax.ShapeDtypeStruct((N, PE), _f32),
                 jax.ShapeDtypeStruct((N, D), _f32)],
  )(pe, raw, h, hp, pp, dinv, corr, wck, bck, tk, hid)


def kernel(node_feat, pos_enc, edge_index, W1, b1, W2, b2, Wpe, bpe, temp,
           Wcoef, bcoef, Wcor, bcor):
  row = edge_index[0]
  col = edge_index[1]
  padw = EPW - EW
  rowp = jnp.concatenate(
      [row.reshape(NW, EW), jnp.zeros((NW, padw), jnp.int32)], axis=1)
  dump_cols = jnp.broadcast_to(
      N + (jnp.arange(padw, dtype=jnp.int32) % (NA - N)), (NW, padw))
  colp = jnp.concatenate([col.reshape(NW, EW), dump_cols], axis=1)
  rc3 = jnp.stack(
      [rowp.reshape(NW, G, C), colp.reshape(NW, G, C)], axis=2)
  z128 = jnp.zeros((NA, D), _f32)
  z16 = jnp.zeros((NA, PE), _f32)

  dp = _prop_pe(rc3, z16).reshape(NC, N, PE)

  h, pe, hs, pes, dinv, hidden = _prologue(
      node_feat, W1, b1.reshape(1, D), W2, b2.reshape(1, D),
      pos_enc, Wpe, bpe.reshape(1, PE), dp,
      Wcoef[0].reshape(PE, 1), bcoef[0].reshape(1, 1), temp[0].reshape(1, 1))
  raw_pe = pe

  for k in range(K):
    hp, pp = _prop_both(hs, pes, rc3, z128, z16)
    corr = _flash_tc(pe, Wcor, bcor.reshape(1, PE))
    hp = hp.reshape(NC, N, D)
    pp = pp.reshape(NC, N, PE)
    h, pe, hs, pes, hidden = _iter_tc(
        pe, raw_pe, h, hp, pp, dinv, corr,
        Wcoef[k + 1].reshape(PE, 1), bcoef[k + 1].reshape(1, 1),
        temp[k + 1].reshape(1, 1), hidden)
  return hidden, pe
